# baseline (speedup 1.0000x reference)
import functools

import jax
import jax.numpy as jnp
from jax import lax
from jax.experimental import pallas as pl
from jax.experimental.pallas import tpu as pltpu
from jax.experimental.pallas import tpu_sc as plsc

_NC = 2
_NS = 16
_NW = _NC * _NS
_CS = 100
_CB = 4
_NBUF = 3


def _pad_rows(n):
    step = 8 * _NS
    return ((n + step - 1) // step) * step


def _sc_mesh():
    return plsc.VectorSubcoreMesh(core_axis_name="c", subcore_axis_name="s")


def _degree_sc(dst, n, d):
    e = dst.shape[0]
    epw = e // _NW
    npad = _pad_rows(n)
    hrows = npad // 128
    vpr = 5
    irows = epw // (16 * vpr)
    assert epw == irows * 16 * vpr

    @functools.partial(
        pl.kernel,
        mesh=_sc_mesh(),
        out_type=jax.ShapeDtypeStruct((_NW * npad,), jnp.float32),
        scratch_types=[
            pltpu.VMEM((irows, 16 * vpr), jnp.int32),
            pltpu.VMEM((npad,), jnp.float32),
        ],
        compiler_params=pltpu.CompilerParams(needs_layout_passes=False),
    )
    def k(dst_hbm, zeros_hbm, out_hbm, didx, hist):
        cid = lax.axis_index("c")
        sid = lax.axis_index("s")
        w = cid * _NS + sid

        pltpu.sync_copy(dst_hbm.at[w], didx)
        pltpu.sync_copy(zeros_hbm, hist)

        ones = jnp.full((16,), 1.0, jnp.float32)

        @pl.loop(0, irows)
        def _(j):
            for u in range(vpr):
                v = didx[j, pl.ds(u * 16, 16)]
                plsc.addupdate_scatter(hist, [v], ones)

        pltpu.sync_copy(hist, out_hbm.at[pl.ds(w * npad, npad)])

    dst3 = dst.reshape(_NW, irows, 16 * vpr)
    zeros1 = jnp.zeros((npad,), jnp.float32)
    deg1 = k(dst3, zeros1)
    return deg1.reshape(_NW, hrows, 128)


def _segsum_sc(g, src, dst, zeros, n):
    e = src.shape[0]
    d = g.shape[1]
    epw = e // _NW
    nchunk = epw // _CS
    nblk = nchunk // _CB
    npad = _pad_rows(n)
    rpw = npad // _NS
    assert e == epw * _NW and epw == nchunk * _CS and nchunk == nblk * _CB
    assert _CB % _NBUF == 1

    @functools.partial(
        pl.kernel,
        mesh=_sc_mesh(),
        out_type=jax.ShapeDtypeStruct((_NC, npad, d), jnp.float32),
        scratch_types=[
            pltpu.VMEM((_CB, _CS), jnp.int32),
            pltpu.VMEM((_CB, _CS), jnp.int32),
        ] + [pltpu.VMEM((_CS, d), jnp.float32)] * _NBUF
        + [pltpu.VMEM_SHARED((npad, d), jnp.float32)]
        + [pltpu.SemaphoreType.DMA] * (2 * _NBUF),
    )
    def k(g_hbm, src_hbm, dst_hbm, zeros_hbm, out_hbm, sidx, didx, *rest):
        rows = rest[:_NBUF]
        acc = rest[_NBUF]
        semg = rest[_NBUF + 1:2 * _NBUF + 1]
        sems = rest[2 * _NBUF + 1:]
        cid = lax.axis_index("c")
        sid = lax.axis_index("s")
        w = cid * _NS + sid

        pltpu.sync_copy(zeros_hbm.at[pl.ds(sid * rpw, rpw)],
                        acc.at[pl.ds(sid * rpw, rpw)])
        plsc.subcore_barrier()

        def gather(c, u):
            pltpu.async_copy(g_hbm.at[sidx.at[c]], rows[u], semg[u])

        def scat(c, u):
            pltpu.async_copy(rows[u], acc.at[didx.at[c]], sems[u], add=True)

        def g_wait(u):
            pltpu.make_async_copy(g_hbm.at[sidx.at[0]], rows[u],
                                  semg[u]).wait()

        def s_wait(u):
            pltpu.make_async_copy(rows[u], acc.at[didx.at[0]],
                                  sems[u]).wait()

        nsteady = (_CB - _NBUF - 1) // _NBUF
        for b in range(nblk):
            pltpu.sync_copy(src_hbm.at[w, b], sidx)
            pltpu.sync_copy(dst_hbm.at[w, b], didx)
            for u in range(_NBUF):
                gather(u, u)

            @pl.loop(0, nsteady)
            def _(j):
                c = _NBUF * j
                for u in range(_NBUF):
                    g_wait(u)
                    scat(c + u, u)
                for u in range(_NBUF):
                    s_wait(u)
                    gather(c + _NBUF + u, u)

            c0 = _NBUF * nsteady
            for u in range(_NBUF):
                g_wait(u)
                scat(c0 + u, u)
            s_wait(0)
            gather(_CB - 1, 0)
            g_wait(0)
            scat(_CB - 1, 0)
            for u in range(_NBUF):
                s_wait(u)

        plsc.subcore_barrier()
        pltpu.sync_copy(acc.at[pl.ds(sid * rpw, rpw)],
                        out_hbm.at[cid, pl.ds(sid * rpw, rpw)])

    src4 = src.reshape(_NW, nblk, _CB, _CS)
    dst4 = dst.reshape(_NW, nblk, _CB, _CS)
    return k(g, src4, dst4, zeros)


def _deg_field(deg_ref, bn, d):
    s = jnp.sum(deg_ref[...], axis=0)
    t = s.T
    parts = [jnp.broadcast_to(t[:, r:r + 1], (128, d))
             for r in range(bn // 128)]
    field = jnp.concatenate(parts, axis=0)
    return jnp.maximum(field, 1.0)


def _scale_tc(x, degp, n, d, bn=1024):
    hb = bn // 128

    def body(x_ref, deg_ref, o_ref):
        d0 = _deg_field(deg_ref, bn, d)
        o_ref[...] = x_ref[...] * lax.rsqrt(d0)

    return pl.pallas_call(
        body,
        grid=(pl.cdiv(n, bn),),
        in_specs=[
            pl.BlockSpec((bn, d), lambda i: (i, 0)),
            pl.BlockSpec((_NW, hb, 128), lambda i: (0, i, 0)),
        ],
        out_specs=pl.BlockSpec((bn, d), lambda i: (i, 0)),
        out_shape=jax.ShapeDtypeStruct((n, d), jnp.float32),
    )(x, degp)


def _rescale_tc(sp, degp, n, d, bn=1024):
    hb = bn // 128

    def body(s_ref, deg_ref, o_ref):
        d0 = _deg_field(deg_ref, bn, d)
        s = s_ref[0] + s_ref[1]
        o_ref[...] = s / d0

    return pl.pallas_call(
        body,
        grid=(pl.cdiv(n, bn),),
        in_specs=[
            pl.BlockSpec((_NC, bn, d), lambda i: (0, i, 0)),
            pl.BlockSpec((_NW, hb, 128), lambda i: (0, i, 0)),
        ],
        out_specs=pl.BlockSpec((bn, d), lambda i: (i, 0)),
        out_shape=jax.ShapeDtypeStruct((n, d), jnp.float32),
    )(sp, degp)


def _combine_tc(x, s1p, s2p, degp, w, bpad, apad, n, d, bn=1024):
    hb = bn // 128

    def body(x_ref, s1_ref, s2_ref, deg_ref, w_ref, b_ref, a_ref, o_ref):
        d0 = _deg_field(deg_ref, bn, d)
        norm = lax.rsqrt(d0)
        a0 = a_ref[0, 0]
        a1 = a_ref[0, 1]
        a2 = a_ref[0, 2]
        s1 = s1_ref[0] + s1_ref[1]
        s2 = s2_ref[0] + s2_ref[1]
        combo = a0 * x_ref[...] + norm * (a1 * s1 + a2 * s2)
        acc = lax.dot_general(
            combo, w_ref[...],
            (((1,), (1,)), ((), ())),
            preferred_element_type=jnp.float32,
            precision=lax.Precision.HIGHEST,
        )
        o_ref[...] = acc + 3.0 * b_ref[...]

    return pl.pallas_call(
        body,
        grid=(pl.cdiv(n, bn),),
        in_specs=[
            pl.BlockSpec((bn, d), lambda i: (i, 0)),
            pl.BlockSpec((_NC, bn, d), lambda i: (0, i, 0)),
            pl.BlockSpec((_NC, bn, d), lambda i: (0, i, 0)),
            pl.BlockSpec((_NW, hb, 128), lambda i: (0, i, 0)),
            pl.BlockSpec((d, d), lambda i: (0, 0)),
            pl.BlockSpec((1, d), lambda i: (0, 0)),
            pl.BlockSpec((1, d), lambda i: (0, 0)),
        ],
        out_specs=pl.BlockSpec((bn, d), lambda i: (i, 0)),
        out_shape=jax.ShapeDtypeStruct((n, d), jnp.float32),
    )(x, s1p, s2p, degp, w, bpad, apad)


def kernel(x, edge_index, W, b, alpha):
    n, d = x.shape
    src = edge_index[0]
    dst = edge_index[1]

    npad = _pad_rows(n)
    zeros = jnp.zeros((npad, d), jnp.float32)
    bpad = b.reshape(1, d).astype(jnp.float32)
    apad = jnp.zeros((1, d), jnp.float32).at[0, :3].set(alpha)

    degp = _degree_sc(dst, n, d)
    g0 = _scale_tc(x.astype(jnp.float32), degp, n, d)
    s1p = _segsum_sc(g0, src, dst, zeros, n)
    g1 = _rescale_tc(s1p, degp, n, d)
    s2p = _segsum_sc(g1, src, dst, zeros, n)
    return _combine_tc(x.astype(jnp.float32), s1p, s2p, degp,
                       W.astype(jnp.float32), bpad, apad, n, d)


# speedup vs baseline: 1.3269x; 1.3269x over previous
import functools

import jax
import jax.numpy as jnp
from jax import lax
from jax.experimental import pallas as pl
from jax.experimental.pallas import tpu as pltpu
from jax.experimental.pallas import tpu_sc as plsc

_NC = 2
_NS = 16
_NW = _NC * _NS
_CS = 80
_CB = 25
_NBUF = 4


def _pad_rows(n):
    step = 8 * _NS
    return ((n + step - 1) // step) * step


def _sc_mesh():
    return plsc.VectorSubcoreMesh(core_axis_name="c", subcore_axis_name="s")


def _degree_sc(dst, n, d):
    e = dst.shape[0]
    epw = e // _NW
    npad = _pad_rows(n)
    hrows = npad // 128
    vpr = 5
    irows = epw // (16 * vpr)
    assert epw == irows * 16 * vpr

    @functools.partial(
        pl.kernel,
        mesh=_sc_mesh(),
        out_type=jax.ShapeDtypeStruct((_NW * npad,), jnp.float32),
        scratch_types=[
            pltpu.VMEM((irows, 16 * vpr), jnp.int32),
            pltpu.VMEM((npad,), jnp.float32),
        ],
        compiler_params=pltpu.CompilerParams(needs_layout_passes=False),
    )
    def k(dst_hbm, zeros_hbm, out_hbm, didx, hist):
        cid = lax.axis_index("c")
        sid = lax.axis_index("s")
        w = cid * _NS + sid

        pltpu.sync_copy(dst_hbm.at[w], didx)
        pltpu.sync_copy(zeros_hbm, hist)

        ones = jnp.full((16,), 1.0, jnp.float32)

        @pl.loop(0, irows)
        def _(j):
            for u in range(vpr):
                v = didx[j, pl.ds(u * 16, 16)]
                plsc.addupdate_scatter(hist, [v], ones)

        pltpu.sync_copy(hist, out_hbm.at[pl.ds(w * npad, npad)])

    dst3 = dst.reshape(_NW, irows, 16 * vpr)
    zeros1 = jnp.zeros((npad,), jnp.float32)
    deg1 = k(dst3, zeros1)
    return deg1.reshape(_NW, hrows, 128)


def _segsum_sc(g, src, dst, zeros, n):
    e = src.shape[0]
    d = g.shape[1]
    epw = e // _NW
    nchunk = epw // _CS
    nblk = nchunk // _CB
    npad = _pad_rows(n)
    rpw = npad // _NS
    assert e == epw * _NW and epw == nchunk * _CS and nchunk == nblk * _CB
    assert _CB % _NBUF == 1

    @functools.partial(
        pl.kernel,
        mesh=_sc_mesh(),
        out_type=jax.ShapeDtypeStruct((_NC, npad, d), jnp.float32),
        scratch_types=[
            pltpu.VMEM((_CB, _CS), jnp.int32),
            pltpu.VMEM((_CB, _CS), jnp.int32),
        ] + [pltpu.VMEM((_CS, d), jnp.float32)] * _NBUF
        + [pltpu.VMEM_SHARED((npad, d), jnp.float32)]
        + [pltpu.SemaphoreType.DMA] * (2 * _NBUF),
    )
    def k(g_hbm, src_hbm, dst_hbm, zeros_hbm, out_hbm, sidx, didx, *rest):
        rows = rest[:_NBUF]
        acc = rest[_NBUF]
        semg = rest[_NBUF + 1:2 * _NBUF + 1]
        sems = rest[2 * _NBUF + 1:]
        cid = lax.axis_index("c")
        sid = lax.axis_index("s")
        w = cid * _NS + sid

        pltpu.sync_copy(zeros_hbm.at[pl.ds(sid * rpw, rpw)],
                        acc.at[pl.ds(sid * rpw, rpw)])
        plsc.subcore_barrier()

        def gather(c, u):
            pltpu.async_copy(g_hbm.at[sidx.at[c]], rows[u], semg[u])

        def scat(c, u):
            pltpu.async_copy(rows[u], acc.at[didx.at[c]], sems[u], add=True)

        def g_wait(u):
            pltpu.make_async_copy(g_hbm.at[sidx.at[0]], rows[u],
                                  semg[u]).wait()

        def s_wait(u):
            pltpu.make_async_copy(rows[u], acc.at[didx.at[0]],
                                  sems[u]).wait()

        nsteady = (_CB - _NBUF - 1) // _NBUF
        for b in range(nblk):
            pltpu.sync_copy(src_hbm.at[w, b], sidx)
            pltpu.sync_copy(dst_hbm.at[w, b], didx)
            for u in range(_NBUF):
                gather(u, u)

            @pl.loop(0, nsteady)
            def _(j):
                c = _NBUF * j
                for u in range(_NBUF):
                    g_wait(u)
                    scat(c + u, u)
                for u in range(_NBUF):
                    s_wait(u)
                    gather(c + _NBUF + u, u)

            c0 = _NBUF * nsteady
            for u in range(_NBUF):
                g_wait(u)
                scat(c0 + u, u)
            s_wait(0)
            gather(_CB - 1, 0)
            g_wait(0)
            scat(_CB - 1, 0)
            for u in range(_NBUF):
                s_wait(u)

        plsc.subcore_barrier()
        pltpu.sync_copy(acc.at[pl.ds(sid * rpw, rpw)],
                        out_hbm.at[cid, pl.ds(sid * rpw, rpw)])

    src4 = src.reshape(_NW, nblk, _CB, _CS)
    dst4 = dst.reshape(_NW, nblk, _CB, _CS)
    return k(g, src4, dst4, zeros)


def _deg_field(deg_ref, bn, d):
    s = jnp.sum(deg_ref[...], axis=0)
    t = s.T
    parts = [jnp.broadcast_to(t[:, r:r + 1], (128, d))
             for r in range(bn // 128)]
    field = jnp.concatenate(parts, axis=0)
    return jnp.maximum(field, 1.0)


def _scale_tc(x, degp, n, d, bn=1024):
    hb = bn // 128

    def body(x_ref, deg_ref, o_ref):
        d0 = _deg_field(deg_ref, bn, d)
        o_ref[...] = x_ref[...] * lax.rsqrt(d0)

    return pl.pallas_call(
        body,
        grid=(pl.cdiv(n, bn),),
        in_specs=[
            pl.BlockSpec((bn, d), lambda i: (i, 0)),
            pl.BlockSpec((_NW, hb, 128), lambda i: (0, i, 0)),
        ],
        out_specs=pl.BlockSpec((bn, d), lambda i: (i, 0)),
        out_shape=jax.ShapeDtypeStruct((n, d), jnp.float32),
    )(x, degp)


def _rescale_tc(sp, degp, n, d, bn=1024):
    hb = bn // 128

    def body(s_ref, deg_ref, o_ref):
        d0 = _deg_field(deg_ref, bn, d)
        s = s_ref[0] + s_ref[1]
        o_ref[...] = s / d0

    return pl.pallas_call(
        body,
        grid=(pl.cdiv(n, bn),),
        in_specs=[
            pl.BlockSpec((_NC, bn, d), lambda i: (0, i, 0)),
            pl.BlockSpec((_NW, hb, 128), lambda i: (0, i, 0)),
        ],
        out_specs=pl.BlockSpec((bn, d), lambda i: (i, 0)),
        out_shape=jax.ShapeDtypeStruct((n, d), jnp.float32),
    )(sp, degp)


def _combine_tc(x, s1p, s2p, degp, w, bpad, apad, n, d, bn=1024):
    hb = bn // 128

    def body(x_ref, s1_ref, s2_ref, deg_ref, w_ref, b_ref, a_ref, o_ref):
        d0 = _deg_field(deg_ref, bn, d)
        norm = lax.rsqrt(d0)
        a0 = a_ref[0, 0]
        a1 = a_ref[0, 1]
        a2 = a_ref[0, 2]
        s1 = s1_ref[0] + s1_ref[1]
        s2 = s2_ref[0] + s2_ref[1]
        combo = a0 * x_ref[...] + norm * (a1 * s1 + a2 * s2)
        acc = lax.dot_general(
            combo, w_ref[...],
            (((1,), (1,)), ((), ())),
            preferred_element_type=jnp.float32,
            precision=lax.Precision.HIGHEST,
        )
        o_ref[...] = acc + 3.0 * b_ref[...]

    return pl.pallas_call(
        body,
        grid=(pl.cdiv(n, bn),),
        in_specs=[
            pl.BlockSpec((bn, d), lambda i: (i, 0)),
            pl.BlockSpec((_NC, bn, d), lambda i: (0, i, 0)),
            pl.BlockSpec((_NC, bn, d), lambda i: (0, i, 0)),
            pl.BlockSpec((_NW, hb, 128), lambda i: (0, i, 0)),
            pl.BlockSpec((d, d), lambda i: (0, 0)),
            pl.BlockSpec((1, d), lambda i: (0, 0)),
            pl.BlockSpec((1, d), lambda i: (0, 0)),
        ],
        out_specs=pl.BlockSpec((bn, d), lambda i: (i, 0)),
        out_shape=jax.ShapeDtypeStruct((n, d), jnp.float32),
    )(x, s1p, s2p, degp, w, bpad, apad)


def kernel(x, edge_index, W, b, alpha):
    n, d = x.shape
    src = edge_index[0]
    dst = edge_index[1]

    npad = _pad_rows(n)
    zeros = jnp.zeros((npad, d), jnp.float32)
    bpad = b.reshape(1, d).astype(jnp.float32)
    apad = jnp.zeros((1, d), jnp.float32).at[0, :3].set(alpha)

    degp = _degree_sc(dst, n, d)
    g0 = _scale_tc(x.astype(jnp.float32), degp, n, d)
    s1p = _segsum_sc(g0, src, dst, zeros, n)
    g1 = _rescale_tc(s1p, degp, n, d)
    s2p = _segsum_sc(g1, src, dst, zeros, n)
    return _combine_tc(x.astype(jnp.float32), s1p, s2p, degp,
                       W.astype(jnp.float32), bpad, apad, n, d)
